# Initial kernel scaffold; baseline (speedup 1.0000x reference)
import jax
import jax.numpy as jnp
from jax.experimental import pallas as pl


def kernel(tokens, word_table, pos_table):
    raise NotImplementedError("write your pallas kernel here")


# speedup vs baseline: 1.8797x; 1.8797x over previous
import jax
import jax.numpy as jnp
from jax import lax
from jax.experimental import pallas as pl
from jax.experimental.pallas import tpu as pltpu
from jax.experimental.pallas import tpu_sc as plsc

LANES = 16
CHUNK = 128


def kernel(tokens, word_table, pos_table):
    B, S = tokens.shape
    V, D = word_table.shape
    N = B * S

    info = plsc.get_sparse_core_info()
    NC, NS = info.num_cores, info.num_subcores
    NW = NC * NS

    n_per_w = N // NW
    chunks = n_per_w // CHUNK
    n_cols = D // LANES

    tokens_1d = tokens.reshape(N).astype(jnp.int32)

    mesh = plsc.VectorSubcoreMesh(core_axis_name="c", subcore_axis_name="s")

    def body(tok_hbm, word_hbm, pos_hbm, out_hbm, idx_v, pos_v, rows_v, sem):
        wid = lax.axis_index("s") * NC + lax.axis_index("c")
        base = wid * n_per_w

        pltpu.sync_copy(tok_hbm.at[pl.ds(base, n_per_w)], idx_v)
        pltpu.sync_copy(pos_hbm, pos_v.at[pl.ds(0, S)])
        pltpu.sync_copy(pos_hbm, pos_v.at[pl.ds(S, S)])

        def chunk_body(k, carry):
            idx_sl = idx_v.at[pl.ds(k * CHUNK, CHUNK)]
            pltpu.async_copy(word_hbm.at[idx_sl], rows_v, sem).wait()
            p_off = lax.rem(k * CHUNK, S)

            def row_body(r, c2):
                p = p_off + r
                for cc in range(n_cols):
                    sl = pl.ds(cc * LANES, LANES)
                    rows_v[r, sl] = rows_v[r, sl] + pos_v[p, sl]
                return c2

            lax.fori_loop(0, CHUNK, row_body, 0)
            pltpu.sync_copy(rows_v, out_hbm.at[pl.ds(base + k * CHUNK, CHUNK)])
            return carry

        lax.fori_loop(0, chunks, chunk_body, 0)

    f = pl.kernel(
        body,
        out_type=jax.ShapeDtypeStruct((N, D), jnp.float32),
        mesh=mesh,
        scratch_types=[
            pltpu.VMEM((n_per_w,), jnp.int32),
            pltpu.VMEM((2 * S, D), jnp.float32),
            pltpu.VMEM((CHUNK, D), jnp.float32),
            pltpu.SemaphoreType.DMA,
        ],
    )
    out = f(tokens_1d, word_table, pos_table)
    return out.reshape(B, S, D)


# speedup vs baseline: 2.5304x; 1.3462x over previous
import jax
import jax.numpy as jnp
from jax import lax
from jax.experimental import pallas as pl
from jax.experimental.pallas import tpu as pltpu
from jax.experimental.pallas import tpu_sc as plsc

LANES = 16
CHUNK = 64
NBUF = 4


def kernel(tokens, word_table, pos_table):
    B, S = tokens.shape
    V, D = word_table.shape
    N = B * S

    info = plsc.get_sparse_core_info()
    NC, NS = info.num_cores, info.num_subcores
    NW = NC * NS

    n_per_w = N // NW
    chunks = n_per_w // CHUNK
    groups = chunks // NBUF
    n_cols = D // LANES

    tokens_1d = tokens.reshape(N).astype(jnp.int32)

    mesh = plsc.VectorSubcoreMesh(core_axis_name="c", subcore_axis_name="s")

    def body(tok_hbm, word_hbm, pos_hbm, out_hbm, idx_v, pos_v,
             b0, b1, b2, b3, g0, g1, g2, g3, s0, s1, s2, s3):
        bufs = (b0, b1, b2, b3)
        gsems = (g0, g1, g2, g3)
        ssems = (s0, s1, s2, s3)

        wid = lax.axis_index("s") * NC + lax.axis_index("c")
        base = wid * n_per_w

        def idx_sl(k):
            return idx_v.at[pl.ds(k * CHUNK, CHUNK)]

        def out_sl(k):
            return out_hbm.at[pl.ds(base + k * CHUNK, CHUNK)]

        pltpu.sync_copy(tok_hbm.at[pl.ds(base, n_per_w)], idx_v)
        for b in range(2):
            pltpu.async_copy(word_hbm.at[idx_sl(b)], bufs[b], gsems[b])
        pltpu.sync_copy(pos_hbm, pos_v.at[pl.ds(0, S)])
        pltpu.sync_copy(pos_hbm, pos_v.at[pl.ds(S, S)])

        def add_rows(buf, k):
            p_off = lax.rem(k * CHUNK, S)

            def row_body(r, c2):
                p = p_off + r
                for cc in range(n_cols):
                    sl = pl.ds(cc * LANES, LANES)
                    buf[r, sl] = buf[r, sl] + pos_v[p, sl]
                return c2

            lax.fori_loop(0, CHUNK, row_body, 0)

        def group(i, carry):
            for b in range(NBUF):
                k = i * NBUF + b
                nb = (b + 2) % NBUF

                @pl.when(k + 2 < chunks)
                def _():
                    @pl.when(k + 2 - NBUF >= 0)
                    def _():
                        pltpu.make_async_copy(
                            bufs[nb], out_sl(k + 2 - NBUF), ssems[nb]).wait()
                    pltpu.async_copy(word_hbm.at[idx_sl(k + 2)], bufs[nb], gsems[nb])

                pltpu.make_async_copy(word_hbm.at[idx_sl(k)], bufs[b], gsems[b]).wait()
                add_rows(bufs[b], k)
                pltpu.async_copy(bufs[b], out_sl(k), ssems[b])
            return carry

        lax.fori_loop(0, groups, group, 0)
        for j in range(chunks - NBUF, chunks):
            pltpu.make_async_copy(bufs[j % NBUF], out_sl(j), ssems[j % NBUF]).wait()

    f = pl.kernel(
        body,
        out_type=jax.ShapeDtypeStruct((N, D), jnp.float32),
        mesh=mesh,
        scratch_types=(
            [pltpu.VMEM((n_per_w,), jnp.int32),
             pltpu.VMEM((2 * S, D), jnp.float32)]
            + [pltpu.VMEM((CHUNK, D), jnp.float32) for _ in range(NBUF)]
            + [pltpu.SemaphoreType.DMA for _ in range(2 * NBUF)]
        ),
    )
    out = f(tokens_1d, word_table, pos_table)
    return out.reshape(B, S, D)


# speedup vs baseline: 7.2894x; 2.8807x over previous
import jax
import jax.numpy as jnp
from jax import lax
from jax.experimental import pallas as pl
from jax.experimental.pallas import tpu as pltpu
from jax.experimental.pallas import tpu_sc as plsc

LANES = 16
CHUNK = 64
NBUF = 4


def kernel(tokens, word_table, pos_table):
    B, S = tokens.shape
    V, D = word_table.shape
    N = B * S

    info = plsc.get_sparse_core_info()
    NC, NS = info.num_cores, info.num_subcores
    NW = NC * NS

    n_per_w = N // NW
    chunks = n_per_w // CHUNK

    tokens_1d = tokens.reshape(N).astype(jnp.int32)

    mesh = plsc.VectorSubcoreMesh(core_axis_name="c", subcore_axis_name="s")

    def body(tok_hbm, word_hbm, pos_hbm, out_hbm, idx_v, pos_sh,
             b0, b1, b2, b3, p0, p1, p2, p3, g0, g1, g2, g3, s0, s1, s2, s3):
        bufs = (b0, b1, b2, b3)
        psems = (p0, p1, p2, p3)
        gsems = (g0, g1, g2, g3)
        ssems = (s0, s1, s2, s3)

        sid = lax.axis_index("s")
        wid = sid * NC + lax.axis_index("c")
        base = wid * n_per_w

        def idx_sl(k):
            return idx_v.at[pl.ds(k * CHUNK, CHUNK)]

        def out_sl(k):
            return out_hbm.at[pl.ds(base + k * CHUNK, CHUNK)]

        def pos_sl(k):
            return pos_sh.at[pl.ds(lax.rem(k * CHUNK, S), CHUNK)]

        def prefill(k, b):
            pltpu.async_copy(pos_sl(k), bufs[b], psems[b])

        def gather_add(k, b):
            pltpu.async_copy(word_hbm.at[idx_sl(k)], bufs[b], gsems[b], add=True)

        @pl.when(sid == 0)
        def _():
            pltpu.sync_copy(pos_hbm, pos_sh.at[pl.ds(0, S)])
            pltpu.sync_copy(pos_hbm, pos_sh.at[pl.ds(S, S)])
        pltpu.sync_copy(tok_hbm.at[pl.ds(base, n_per_w)], idx_v)
        plsc.subcore_barrier()

        prefill(0, 0)
        prefill(1, 1)
        pltpu.make_async_copy(pos_sl(0), bufs[0], psems[0]).wait()
        gather_add(0, 0)

        def group(i, carry):
            for b in range(NBUF):
                k = i * NBUF + b
                b2 = (b + 2) % NBUF
                b1 = (b + 1) % NBUF

                @pl.when(k + 2 < chunks)
                def _():
                    @pl.when(k - 2 >= 0)
                    def _():
                        pltpu.make_async_copy(
                            bufs[b2], out_sl(k - 2), ssems[b2]).wait()
                    prefill(k + 2, b2)

                @pl.when(k + 1 < chunks)
                def _():
                    pltpu.make_async_copy(
                        pos_sl(k + 1), bufs[b1], psems[b1]).wait()
                    gather_add(k + 1, b1)

                pltpu.make_async_copy(
                    word_hbm.at[idx_sl(k)], bufs[b], gsems[b]).wait()
                pltpu.async_copy(bufs[b], out_sl(k), ssems[b])
            return carry

        lax.fori_loop(0, chunks // NBUF, group, 0)
        for j in range(chunks - NBUF, chunks):
            pltpu.make_async_copy(bufs[j % NBUF], out_sl(j), ssems[j % NBUF]).wait()

    f = pl.kernel(
        body,
        out_type=jax.ShapeDtypeStruct((N, D), jnp.float32),
        mesh=mesh,
        scratch_types=(
            [pltpu.VMEM((n_per_w,), jnp.int32),
             pltpu.VMEM_SHARED((2 * S, D), jnp.float32)]
            + [pltpu.VMEM((CHUNK, D), jnp.float32) for _ in range(NBUF)]
            + [pltpu.SemaphoreType.DMA for _ in range(3 * NBUF)]
        ),
    )
    out = f(tokens_1d, word_table, pos_table)
    return out.reshape(B, S, D)


# speedup vs baseline: 7.6042x; 1.0432x over previous
import jax
import jax.numpy as jnp
from jax import lax
from jax.experimental import pallas as pl
from jax.experimental.pallas import tpu as pltpu
from jax.experimental.pallas import tpu_sc as plsc

LANES = 16
CHUNK = 80
NBUF = 4


def kernel(tokens, word_table, pos_table):
    B, S = tokens.shape
    V, D = word_table.shape
    N = B * S

    info = plsc.get_sparse_core_info()
    NC, NS = info.num_cores, info.num_subcores
    NW = NC * NS

    n_per_w = N // NW
    chunks = n_per_w // CHUNK

    tokens_1d = tokens.reshape(N).astype(jnp.int32)

    mesh = plsc.VectorSubcoreMesh(core_axis_name="c", subcore_axis_name="s")

    def body(tok_hbm, word_hbm, pos_hbm, out_hbm, idx_v, pos_sh,
             b0, b1, b2, b3, p0, p1, p2, p3, g0, g1, g2, g3, s0, s1, s2, s3):
        bufs = (b0, b1, b2, b3)
        psems = (p0, p1, p2, p3)
        gsems = (g0, g1, g2, g3)
        ssems = (s0, s1, s2, s3)

        sid = lax.axis_index("s")
        wid = sid * NC + lax.axis_index("c")
        base = wid * n_per_w

        def idx_sl(k):
            return idx_v.at[pl.ds(k * CHUNK, CHUNK)]

        def out_sl(k):
            return out_hbm.at[pl.ds(base + k * CHUNK, CHUNK)]

        def pos_sl(k):
            return pos_sh.at[pl.ds(lax.rem(k * CHUNK, S), CHUNK)]

        def prefill(k, b):
            pltpu.async_copy(pos_sl(k), bufs[b], psems[b])

        def gather_add(k, b):
            pltpu.async_copy(word_hbm.at[idx_sl(k)], bufs[b], gsems[b], add=True)

        @pl.when(sid == 0)
        def _():
            pltpu.sync_copy(pos_hbm, pos_sh.at[pl.ds(0, S)])
            pltpu.sync_copy(pos_hbm, pos_sh.at[pl.ds(S, S)])
        pltpu.sync_copy(tok_hbm.at[pl.ds(base, n_per_w)], idx_v)
        plsc.subcore_barrier()

        prefill(0, 0)
        prefill(1, 1)
        pltpu.make_async_copy(pos_sl(0), bufs[0], psems[0]).wait()
        gather_add(0, 0)

        def group(i, carry):
            for b in range(NBUF):
                k = i * NBUF + b
                b2 = (b + 2) % NBUF
                b1 = (b + 1) % NBUF

                @pl.when(k + 2 < chunks)
                def _():
                    @pl.when(k - 2 >= 0)
                    def _():
                        pltpu.make_async_copy(
                            bufs[b2], out_sl(k - 2), ssems[b2]).wait()
                    prefill(k + 2, b2)

                @pl.when(k + 1 < chunks)
                def _():
                    pltpu.make_async_copy(
                        pos_sl(k + 1), bufs[b1], psems[b1]).wait()
                    gather_add(k + 1, b1)

                pltpu.make_async_copy(
                    word_hbm.at[idx_sl(k)], bufs[b], gsems[b]).wait()
                pltpu.async_copy(bufs[b], out_sl(k), ssems[b])
            return carry

        lax.fori_loop(0, chunks // NBUF, group, 0)
        for j in range(chunks - NBUF, chunks):
            pltpu.make_async_copy(bufs[j % NBUF], out_sl(j), ssems[j % NBUF]).wait()

    f = pl.kernel(
        body,
        out_type=jax.ShapeDtypeStruct((N, D), jnp.float32),
        mesh=mesh,
        scratch_types=(
            [pltpu.VMEM((n_per_w,), jnp.int32),
             pltpu.VMEM_SHARED((2 * S, D), jnp.float32)]
            + [pltpu.VMEM((CHUNK, D), jnp.float32) for _ in range(NBUF)]
            + [pltpu.SemaphoreType.DMA for _ in range(3 * NBUF)]
        ),
    )
    out = f(tokens_1d, word_table, pos_table)
    return out.reshape(B, S, D)


# speedup vs baseline: 7.7619x; 1.0207x over previous
import jax
import jax.numpy as jnp
from jax import lax
from jax.experimental import pallas as pl
from jax.experimental.pallas import tpu as pltpu
from jax.experimental.pallas import tpu_sc as plsc

LANES = 16
CHUNK = 128
NBUF = 4


def kernel(tokens, word_table, pos_table):
    B, S = tokens.shape
    V, D = word_table.shape
    N = B * S

    info = plsc.get_sparse_core_info()
    NC, NS = info.num_cores, info.num_subcores
    NW = NC * NS

    n_per_w = N // NW
    chunks = n_per_w // CHUNK

    tokens_1d = tokens.reshape(N).astype(jnp.int32)

    mesh = plsc.VectorSubcoreMesh(core_axis_name="c", subcore_axis_name="s")

    def body(tok_hbm, word_hbm, pos_hbm, out_hbm, idx_v, pos_sh,
             b0, b1, b2, b3, p0, p1, p2, p3, g0, g1, g2, g3, s0, s1, s2, s3):
        bufs = (b0, b1, b2, b3)
        psems = (p0, p1, p2, p3)
        gsems = (g0, g1, g2, g3)
        ssems = (s0, s1, s2, s3)

        sid = lax.axis_index("s")
        wid = sid * NC + lax.axis_index("c")
        base = wid * n_per_w

        def idx_sl(k):
            return idx_v.at[pl.ds(k * CHUNK, CHUNK)]

        def out_sl(k):
            return out_hbm.at[pl.ds(base + k * CHUNK, CHUNK)]

        def pos_sl(k):
            return pos_sh.at[pl.ds(lax.rem(k * CHUNK, S), CHUNK)]

        def prefill(k, b):
            pltpu.async_copy(pos_sl(k), bufs[b], psems[b])

        def gather_add(k, b):
            pltpu.async_copy(word_hbm.at[idx_sl(k)], bufs[b], gsems[b], add=True)

        @pl.when(sid == 0)
        def _():
            pltpu.sync_copy(pos_hbm, pos_sh.at[pl.ds(0, S)])
            pltpu.sync_copy(pos_hbm, pos_sh.at[pl.ds(S, S)])
        pltpu.sync_copy(tok_hbm.at[pl.ds(base, n_per_w)], idx_v)
        plsc.subcore_barrier()

        prefill(0, 0)
        prefill(1, 1)
        pltpu.make_async_copy(pos_sl(0), bufs[0], psems[0]).wait()
        gather_add(0, 0)

        def group(i, carry):
            for b in range(NBUF):
                k = i * NBUF + b
                b2 = (b + 2) % NBUF
                b1 = (b + 1) % NBUF

                @pl.when(k + 2 < chunks)
                def _():
                    @pl.when(k - 2 >= 0)
                    def _():
                        pltpu.make_async_copy(
                            bufs[b2], out_sl(k - 2), ssems[b2]).wait()
                    prefill(k + 2, b2)

                @pl.when(k + 1 < chunks)
                def _():
                    pltpu.make_async_copy(
                        pos_sl(k + 1), bufs[b1], psems[b1]).wait()
                    gather_add(k + 1, b1)

                pltpu.make_async_copy(
                    word_hbm.at[idx_sl(k)], bufs[b], gsems[b]).wait()
                pltpu.async_copy(bufs[b], out_sl(k), ssems[b])
            return carry

        lax.fori_loop(0, chunks // NBUF, group, 0)
        for k in range(chunks - chunks % NBUF, chunks):
            b = k % NBUF
            if k + 1 < chunks:
                b1 = (k + 1) % NBUF
                pltpu.make_async_copy(pos_sl(k + 1), bufs[b1], psems[b1]).wait()
                gather_add(k + 1, b1)
            pltpu.make_async_copy(
                word_hbm.at[idx_sl(k)], bufs[b], gsems[b]).wait()
            pltpu.async_copy(bufs[b], out_sl(k), ssems[b])
        for j in range(chunks - NBUF, chunks):
            pltpu.make_async_copy(bufs[j % NBUF], out_sl(j), ssems[j % NBUF]).wait()

    f = pl.kernel(
        body,
        out_type=jax.ShapeDtypeStruct((N, D), jnp.float32),
        mesh=mesh,
        scratch_types=(
            [pltpu.VMEM((n_per_w,), jnp.int32),
             pltpu.VMEM_SHARED((2 * S, D), jnp.float32)]
            + [pltpu.VMEM((CHUNK, D), jnp.float32) for _ in range(NBUF)]
            + [pltpu.SemaphoreType.DMA for _ in range(3 * NBUF)]
        ),
    )
    out = f(tokens_1d, word_table, pos_table)
    return out.reshape(B, S, D)


# speedup vs baseline: 7.7941x; 1.0041x over previous
import jax
import jax.numpy as jnp
from jax import lax
from jax.experimental import pallas as pl
from jax.experimental.pallas import tpu as pltpu
from jax.experimental.pallas import tpu_sc as plsc

CHUNK = 128
NBUF = 6
PAHEAD = 3
GAHEAD = 2


def kernel(tokens, word_table, pos_table):
    B, S = tokens.shape
    V, D = word_table.shape
    N = B * S

    info = plsc.get_sparse_core_info()
    NC, NS = info.num_cores, info.num_subcores
    NW = NC * NS

    n_per_w = N // NW
    chunks = n_per_w // CHUNK

    tokens_1d = tokens.reshape(N).astype(jnp.int32)

    mesh = plsc.VectorSubcoreMesh(core_axis_name="c", subcore_axis_name="s")

    def body(tok_hbm, word_hbm, pos_hbm, out_hbm, idx_v, pos_sh,
             b0, b1, b2, b3, b4, b5,
             p0, p1, p2, p3, p4, p5,
             g0, g1, g2, g3, g4, g5,
             s0, s1, s2, s3, s4, s5):
        bufs = (b0, b1, b2, b3, b4, b5)
        psems = (p0, p1, p2, p3, p4, p5)
        gsems = (g0, g1, g2, g3, g4, g5)
        ssems = (s0, s1, s2, s3, s4, s5)

        sid = lax.axis_index("s")
        wid = sid * NC + lax.axis_index("c")
        base = wid * n_per_w

        def idx_sl(k):
            return idx_v.at[pl.ds(k * CHUNK, CHUNK)]

        def out_sl(k):
            return out_hbm.at[pl.ds(base + k * CHUNK, CHUNK)]

        def pos_sl(k):
            return pos_sh.at[pl.ds(lax.rem(k * CHUNK, S), CHUNK)]

        def prefill(k, b):
            pltpu.async_copy(pos_sl(k), bufs[b], psems[b])

        def wait_prefill(k, b):
            pltpu.make_async_copy(pos_sl(k), bufs[b], psems[b]).wait()

        def gather_add(k, b):
            pltpu.async_copy(word_hbm.at[idx_sl(k)], bufs[b], gsems[b], add=True)

        def wait_gather(k, b):
            pltpu.make_async_copy(word_hbm.at[idx_sl(k)], bufs[b], gsems[b]).wait()

        def store(k, b):
            pltpu.async_copy(bufs[b], out_sl(k), ssems[b])

        def wait_store(k, b):
            pltpu.make_async_copy(bufs[b], out_sl(k), ssems[b]).wait()

        @pl.when(sid == 0)
        def _():
            pltpu.sync_copy(pos_hbm, pos_sh.at[pl.ds(0, S)])
            pltpu.sync_copy(pos_hbm, pos_sh.at[pl.ds(S, S)])
        pltpu.sync_copy(tok_hbm.at[pl.ds(base, n_per_w)], idx_v)
        plsc.subcore_barrier()

        for j in range(PAHEAD):
            prefill(j, j)
        for j in range(GAHEAD):
            wait_prefill(j, j)
            gather_add(j, j)

        def group(i, carry):
            for b in range(NBUF):
                k = i * NBUF + b
                bp = (b + PAHEAD) % NBUF
                bg = (b + GAHEAD) % NBUF

                @pl.when(k + PAHEAD < chunks)
                def _():
                    @pl.when(k + PAHEAD - NBUF >= 0)
                    def _():
                        wait_store(k + PAHEAD - NBUF, bp)
                    prefill(k + PAHEAD, bp)

                @pl.when(k + GAHEAD < chunks)
                def _():
                    wait_prefill(k + GAHEAD, bg)
                    gather_add(k + GAHEAD, bg)

                wait_gather(k, b)
                store(k, b)
            return carry

        lax.fori_loop(0, chunks // NBUF, group, 0)
        for k in range(chunks - chunks % NBUF, chunks):
            b = k % NBUF
            if k + PAHEAD < chunks:
                bp = (k + PAHEAD) % NBUF
                if k + PAHEAD - NBUF >= 0:
                    wait_store(k + PAHEAD - NBUF, bp)
                prefill(k + PAHEAD, bp)
            if k + GAHEAD < chunks:
                bg = (k + GAHEAD) % NBUF
                wait_prefill(k + GAHEAD, bg)
                gather_add(k + GAHEAD, bg)
            wait_gather(k, b)
            store(k, b)
        for j in range(chunks - NBUF, chunks):
            wait_store(j, j % NBUF)

    f = pl.kernel(
        body,
        out_type=jax.ShapeDtypeStruct((N, D), jnp.float32),
        mesh=mesh,
        scratch_types=(
            [pltpu.VMEM((n_per_w,), jnp.int32),
             pltpu.VMEM_SHARED((2 * S, D), jnp.float32)]
            + [pltpu.VMEM((CHUNK, D), jnp.float32) for _ in range(NBUF)]
            + [pltpu.SemaphoreType.DMA for _ in range(3 * NBUF)]
        ),
    )
    out = f(tokens_1d, word_table, pos_table)
    return out.reshape(B, S, D)


# speedup vs baseline: 7.8183x; 1.0031x over previous
import jax
import jax.numpy as jnp
from jax import lax
from jax.experimental import pallas as pl
from jax.experimental.pallas import tpu as pltpu
from jax.experimental.pallas import tpu_sc as plsc

CHUNK = 128
NBUF = 7
PAHEAD = 4
GAHEAD = 3


def kernel(tokens, word_table, pos_table):
    B, S = tokens.shape
    V, D = word_table.shape
    N = B * S

    info = plsc.get_sparse_core_info()
    NC, NS = info.num_cores, info.num_subcores
    NW = NC * NS

    n_per_w = N // NW
    chunks = n_per_w // CHUNK

    tokens_1d = tokens.reshape(N).astype(jnp.int32)

    mesh = plsc.VectorSubcoreMesh(core_axis_name="c", subcore_axis_name="s")

    def body(tok_hbm, word_hbm, pos_hbm, out_hbm, idx_v, pos_sh,
             b0, b1, b2, b3, b4, b5, b6,
             p0, p1, p2, p3, p4, p5, p6,
             g0, g1, g2, g3, g4, g5, g6,
             s0, s1, s2, s3, s4, s5, s6):
        bufs = (b0, b1, b2, b3, b4, b5, b6)
        psems = (p0, p1, p2, p3, p4, p5, p6)
        gsems = (g0, g1, g2, g3, g4, g5, g6)
        ssems = (s0, s1, s2, s3, s4, s5, s6)

        sid = lax.axis_index("s")
        wid = sid * NC + lax.axis_index("c")
        base = wid * n_per_w

        def idx_sl(k):
            return idx_v.at[pl.ds(k * CHUNK, CHUNK)]

        def out_sl(k):
            return out_hbm.at[pl.ds(base + k * CHUNK, CHUNK)]

        def pos_sl(k):
            return pos_sh.at[pl.ds(lax.rem(k * CHUNK, S), CHUNK)]

        def prefill(k, b):
            pltpu.async_copy(pos_sl(k), bufs[b], psems[b])

        def wait_prefill(k, b):
            pltpu.make_async_copy(pos_sl(k), bufs[b], psems[b]).wait()

        def gather_add(k, b):
            pltpu.async_copy(word_hbm.at[idx_sl(k)], bufs[b], gsems[b], add=True)

        def wait_gather(k, b):
            pltpu.make_async_copy(word_hbm.at[idx_sl(k)], bufs[b], gsems[b]).wait()

        def store(k, b):
            pltpu.async_copy(bufs[b], out_sl(k), ssems[b])

        def wait_store(k, b):
            pltpu.make_async_copy(bufs[b], out_sl(k), ssems[b]).wait()

        @pl.when(sid == 0)
        def _():
            pltpu.sync_copy(pos_hbm, pos_sh.at[pl.ds(0, S)])
            pltpu.sync_copy(pos_hbm, pos_sh.at[pl.ds(S, S)])
        pltpu.sync_copy(tok_hbm.at[pl.ds(base, n_per_w)], idx_v)
        plsc.subcore_barrier()

        for j in range(PAHEAD):
            prefill(j, j)
        for j in range(GAHEAD):
            wait_prefill(j, j)
            gather_add(j, j)

        def group(i, carry):
            for b in range(NBUF):
                k = i * NBUF + b
                bp = (b + PAHEAD) % NBUF
                bg = (b + GAHEAD) % NBUF

                @pl.when(k + PAHEAD < chunks)
                def _():
                    @pl.when(k + PAHEAD - NBUF >= 0)
                    def _():
                        wait_store(k + PAHEAD - NBUF, bp)
                    prefill(k + PAHEAD, bp)

                @pl.when(k + GAHEAD < chunks)
                def _():
                    wait_prefill(k + GAHEAD, bg)
                    gather_add(k + GAHEAD, bg)

                wait_gather(k, b)
                store(k, b)
            return carry

        lax.fori_loop(0, chunks // NBUF, group, 0)
        for k in range(chunks - chunks % NBUF, chunks):
            b = k % NBUF
            if k + PAHEAD < chunks:
                bp = (k + PAHEAD) % NBUF
                if k + PAHEAD - NBUF >= 0:
                    wait_store(k + PAHEAD - NBUF, bp)
                prefill(k + PAHEAD, bp)
            if k + GAHEAD < chunks:
                bg = (k + GAHEAD) % NBUF
                wait_prefill(k + GAHEAD, bg)
                gather_add(k + GAHEAD, bg)
            wait_gather(k, b)
            store(k, b)
        for j in range(chunks - NBUF, chunks):
            wait_store(j, j % NBUF)

    f = pl.kernel(
        body,
        out_type=jax.ShapeDtypeStruct((N, D), jnp.float32),
        mesh=mesh,
        scratch_types=(
            [pltpu.VMEM((n_per_w,), jnp.int32),
             pltpu.VMEM_SHARED((2 * S, D), jnp.float32)]
            + [pltpu.VMEM((CHUNK, D), jnp.float32) for _ in range(NBUF)]
            + [pltpu.SemaphoreType.DMA for _ in range(3 * NBUF)]
        ),
    )
    out = f(tokens_1d, word_table, pos_table)
    return out.reshape(B, S, D)
